# baseline (speedup 1.0000x reference)
import functools

import jax
import jax.numpy as jnp
from jax import lax
from jax.experimental import pallas as pl
from jax.experimental.pallas import tpu as pltpu
from jax.experimental.pallas import tpu_sc as plsc

N = 10000
D = 128
H1 = 64
H2 = 32
E = 320000

NCORES = 2
NSUB = 16
NW = NCORES * NSUB
CHUNK = 128
NCH = 79
NCH0 = 65
NCH1 = 93
EPAD = NW * NCH * CHUNK
NP = 10112
RPT = NP // NSUB
NBLK = 16
MROWS = NP // NBLK


def _spmm_sc(src_c, dst_c, h, zeros):
    F = h.shape[1]
    mesh = plsc.VectorSubcoreMesh(core_axis_name="c", subcore_axis_name="s")

    @functools.partial(
        pl.kernel,
        out_type=jax.ShapeDtypeStruct((NCORES, NP, F), jnp.float32),
        mesh=mesh,
        scratch_types=[
            pltpu.VMEM((NCH1, CHUNK), jnp.int32),
            pltpu.VMEM((NCH1, CHUNK), jnp.int32),
            pltpu.VMEM((CHUNK, F), jnp.float32),
            pltpu.VMEM_SHARED((NP, F), jnp.float32),
        ],
        compiler_params=pltpu.CompilerParams(use_tc_tiling_on_sc=False),
    )
    def spmm(src_hbm, dst_hbm, h_hbm, z_hbm, out_hbm,
             src_v, dst_v, rows_v, acc):
        c = lax.axis_index("c")
        s = lax.axis_index("s")
        w = c * NSUB + s
        row0 = s * RPT
        pltpu.sync_copy(z_hbm.at[pl.ds(row0, RPT)], acc.at[pl.ds(row0, RPT)])
        pltpu.sync_copy(src_hbm.at[w], src_v)
        pltpu.sync_copy(dst_hbm.at[w], dst_v)
        plsc.subcore_barrier()

        nch = jnp.where(c == 0, NCH0, NCH1)

        @pl.loop(0, nch)
        def _(j):
            pltpu.sync_copy(h_hbm.at[src_v.at[j]], rows_v)
            pltpu.sync_copy(rows_v, acc.at[dst_v.at[j]], add=True)

        plsc.subcore_barrier()
        pltpu.sync_copy(acc.at[pl.ds(row0, RPT)],
                        out_hbm.at[c, pl.ds(row0, RPT)])

    return spmm(src_c, dst_c, h, zeros)


def _mm_body(x_ref, w_ref, o_ref):
    o_ref[...] = jnp.dot(x_ref[...], w_ref[...],
                         preferred_element_type=jnp.float32)


def _mid_body(p_ref, w_ref, o_ref):
    h = jnp.maximum(p_ref[0] + p_ref[1], 0.0)
    o_ref[...] = jnp.dot(h, w_ref[...], preferred_element_type=jnp.float32)


def _reparam_body(q_ref, eps_ref, z_ref, mu_ref, lv_ref):
    mu = q_ref[0, :, :H2] + q_ref[1, :, :H2]
    lv = q_ref[0, :, H2:] + q_ref[1, :, H2:]
    mu_ref[...] = mu
    lv_ref[...] = lv
    z_ref[...] = eps_ref[...] * jnp.exp(lv) + mu


def _outer_body(zi_ref, zj_ref, o_ref):
    o_ref[...] = lax.dot_general(zi_ref[...], zj_ref[...],
                                 (((1,), (1,)), ((), ())),
                                 preferred_element_type=jnp.float32)


_RB = 2000
_OB = 400


def kernel(x, edge_index, W1, W2, W3):
    pad = EPAD - E

    def _partition(flat):
        ch = flat.reshape(NW * NCH, CHUNK)
        c0 = ch[:NSUB * NCH0].reshape(NSUB, NCH0, CHUNK)
        c0 = jnp.pad(c0, ((0, 0), (0, NCH1 - NCH0), (0, 0)))
        c1 = ch[NSUB * NCH0:].reshape(NSUB, NCH1, CHUNK)
        return jnp.concatenate([c0, c1], axis=0)

    src_c = _partition(jnp.concatenate(
        [edge_index[0], jnp.zeros((pad,), jnp.int32)]))
    pad_dst = N + (jnp.arange(pad, dtype=jnp.int32) % (NP - N))
    dst_c = _partition(jnp.concatenate([edge_index[1], pad_dst]))
    zeros64 = jnp.zeros((NP, H1), jnp.float32)
    eps = jax.random.normal(jax.random.key(42), (NP, H2), dtype=jnp.float32)
    W23 = jnp.concatenate([W2, W3], axis=1)

    h0 = pl.pallas_call(
        _mm_body,
        grid=(N // _RB,),
        in_specs=[pl.BlockSpec((_RB, D), lambda i: (i, 0)),
                  pl.BlockSpec((D, H1), lambda i: (0, 0))],
        out_specs=pl.BlockSpec((_RB, H1), lambda i: (i, 0)),
        out_shape=jax.ShapeDtypeStruct((N, H1), jnp.float32),
    )(x, W1)

    h0p = jnp.concatenate([h0, jnp.zeros((NP - N, H1), jnp.float32)], axis=0)
    parts1 = _spmm_sc(src_c, dst_c, h0p, zeros64)

    h23 = pl.pallas_call(
        _mid_body,
        grid=(NBLK,),
        in_specs=[pl.BlockSpec((NCORES, MROWS, H1), lambda i: (0, i, 0)),
                  pl.BlockSpec((H1, H1), lambda i: (0, 0))],
        out_specs=pl.BlockSpec((MROWS, H1), lambda i: (i, 0)),
        out_shape=jax.ShapeDtypeStruct((NP, H1), jnp.float32),
    )(parts1, W23)

    parts2 = _spmm_sc(src_c, dst_c, h23, zeros64)

    z_p, mu_p, lv_p = pl.pallas_call(
        _reparam_body,
        grid=(NBLK,),
        in_specs=[pl.BlockSpec((NCORES, MROWS, H1), lambda i: (0, i, 0)),
                  pl.BlockSpec((MROWS, H2), lambda i: (i, 0))],
        out_specs=[pl.BlockSpec((MROWS, H2), lambda i: (i, 0)),
                   pl.BlockSpec((MROWS, H2), lambda i: (i, 0)),
                   pl.BlockSpec((MROWS, H2), lambda i: (i, 0))],
        out_shape=[jax.ShapeDtypeStruct((NP, H2), jnp.float32),
                   jax.ShapeDtypeStruct((NP, H2), jnp.float32),
                   jax.ShapeDtypeStruct((NP, H2), jnp.float32)],
    )(parts2, eps)

    pred_adj = pl.pallas_call(
        _outer_body,
        grid=(N // _OB,),
        in_specs=[pl.BlockSpec((_OB, H2), lambda i: (i, 0)),
                  pl.BlockSpec((N, H2), lambda i: (0, 0))],
        out_specs=pl.BlockSpec((_OB, N), lambda i: (i, 0)),
        out_shape=jax.ShapeDtypeStruct((N, N), jnp.float32),
    )(z_p, z_p)

    return (pred_adj, mu_p[:N], lv_p[:N])


# speedup vs baseline: 1.1111x; 1.1111x over previous
import functools

import jax
import jax.numpy as jnp
from jax import lax
from jax.experimental import pallas as pl
from jax.experimental.pallas import tpu as pltpu
from jax.experimental.pallas import tpu_sc as plsc

N = 10000
D = 128
H1 = 64
H2 = 32
E = 320000

NCORES = 2
NSUB = 16
NW = NCORES * NSUB
CHUNK = 128
NCH = 79
NCH0 = 65
NCH1 = 93
EPAD = NW * NCH * CHUNK
NP = 10112
RPT = NP // NSUB
NBLK = 16
MROWS = NP // NBLK


def _spmm_sc(src_c, dst_c, h, zeros):
    F = h.shape[1]
    mesh = plsc.VectorSubcoreMesh(core_axis_name="c", subcore_axis_name="s")

    @functools.partial(
        pl.kernel,
        out_type=jax.ShapeDtypeStruct((NCORES, NP, F), jnp.float32),
        mesh=mesh,
        scratch_types=[
            pltpu.VMEM((NCH1, CHUNK), jnp.int32),
            pltpu.VMEM((NCH1, CHUNK), jnp.int32),
            pltpu.VMEM((CHUNK, F), jnp.float32),
            pltpu.VMEM_SHARED((NP, F), jnp.float32),
        ],
        compiler_params=pltpu.CompilerParams(use_tc_tiling_on_sc=False),
    )
    def spmm(src_hbm, dst_hbm, h_hbm, z_hbm, out_hbm,
             src_v, dst_v, rows_v, acc):
        c = lax.axis_index("c")
        s = lax.axis_index("s")
        w = c * NSUB + s
        row0 = s * RPT
        pltpu.sync_copy(z_hbm.at[pl.ds(row0, RPT)], acc.at[pl.ds(row0, RPT)])
        pltpu.sync_copy(src_hbm.at[w], src_v)
        pltpu.sync_copy(dst_hbm.at[w], dst_v)
        plsc.subcore_barrier()

        nch = jnp.where(c == 0, NCH1, NCH0)

        @pl.loop(0, nch)
        def _(j):
            pltpu.sync_copy(h_hbm.at[src_v.at[j]], rows_v)
            pltpu.sync_copy(rows_v, acc.at[dst_v.at[j]], add=True)

        plsc.subcore_barrier()
        pltpu.sync_copy(acc.at[pl.ds(row0, RPT)],
                        out_hbm.at[c, pl.ds(row0, RPT)])

    return spmm(src_c, dst_c, h, zeros)


def _mm_body(x_ref, w_ref, o_ref):
    o_ref[...] = jnp.dot(x_ref[...], w_ref[...],
                         preferred_element_type=jnp.float32)


def _mid_body(p_ref, w_ref, o_ref):
    h = jnp.maximum(p_ref[0] + p_ref[1], 0.0)
    o_ref[...] = jnp.dot(h, w_ref[...], preferred_element_type=jnp.float32)


def _reparam_body(q_ref, eps_ref, z_ref, mu_ref, lv_ref):
    mu = q_ref[0, :, :H2] + q_ref[1, :, :H2]
    lv = q_ref[0, :, H2:] + q_ref[1, :, H2:]
    mu_ref[...] = mu
    lv_ref[...] = lv
    z_ref[...] = eps_ref[...] * jnp.exp(lv) + mu


def _outer_body(zi_ref, zj_ref, o_ref):
    o_ref[...] = lax.dot_general(zi_ref[...], zj_ref[...],
                                 (((1,), (1,)), ((), ())),
                                 preferred_element_type=jnp.float32)


_RB = 2000
_OB = 400


def kernel(x, edge_index, W1, W2, W3):
    pad = EPAD - E

    def _partition(flat):
        ch = flat.reshape(NW * NCH, CHUNK)
        c0 = ch[:NSUB * NCH1].reshape(NSUB, NCH1, CHUNK)
        c1 = ch[NSUB * NCH1:].reshape(NSUB, NCH0, CHUNK)
        c1 = jnp.pad(c1, ((0, 0), (0, NCH1 - NCH0), (0, 0)))
        return jnp.concatenate([c0, c1], axis=0)

    src_c = _partition(jnp.concatenate(
        [edge_index[0], jnp.zeros((pad,), jnp.int32)]))
    pad_dst = N + (jnp.arange(pad, dtype=jnp.int32) % (NP - N))
    dst_c = _partition(jnp.concatenate([edge_index[1], pad_dst]))
    zeros64 = jnp.zeros((NP, H1), jnp.float32)
    eps = jax.random.normal(jax.random.key(42), (NP, H2), dtype=jnp.float32)
    W23 = jnp.concatenate([W2, W3], axis=1)

    h0 = pl.pallas_call(
        _mm_body,
        grid=(N // _RB,),
        in_specs=[pl.BlockSpec((_RB, D), lambda i: (i, 0)),
                  pl.BlockSpec((D, H1), lambda i: (0, 0))],
        out_specs=pl.BlockSpec((_RB, H1), lambda i: (i, 0)),
        out_shape=jax.ShapeDtypeStruct((N, H1), jnp.float32),
    )(x, W1)

    h0p = jnp.concatenate([h0, jnp.zeros((NP - N, H1), jnp.float32)], axis=0)
    parts1 = _spmm_sc(src_c, dst_c, h0p, zeros64)

    h23 = pl.pallas_call(
        _mid_body,
        grid=(NBLK,),
        in_specs=[pl.BlockSpec((NCORES, MROWS, H1), lambda i: (0, i, 0)),
                  pl.BlockSpec((H1, H1), lambda i: (0, 0))],
        out_specs=pl.BlockSpec((MROWS, H1), lambda i: (i, 0)),
        out_shape=jax.ShapeDtypeStruct((NP, H1), jnp.float32),
    )(parts1, W23)

    parts2 = _spmm_sc(src_c, dst_c, h23, zeros64)

    z_p, mu_p, lv_p = pl.pallas_call(
        _reparam_body,
        grid=(NBLK,),
        in_specs=[pl.BlockSpec((NCORES, MROWS, H1), lambda i: (0, i, 0)),
                  pl.BlockSpec((MROWS, H2), lambda i: (i, 0))],
        out_specs=[pl.BlockSpec((MROWS, H2), lambda i: (i, 0)),
                   pl.BlockSpec((MROWS, H2), lambda i: (i, 0)),
                   pl.BlockSpec((MROWS, H2), lambda i: (i, 0))],
        out_shape=[jax.ShapeDtypeStruct((NP, H2), jnp.float32),
                   jax.ShapeDtypeStruct((NP, H2), jnp.float32),
                   jax.ShapeDtypeStruct((NP, H2), jnp.float32)],
    )(parts2, eps)

    pred_adj = pl.pallas_call(
        _outer_body,
        grid=(N // _OB,),
        in_specs=[pl.BlockSpec((_OB, H2), lambda i: (i, 0)),
                  pl.BlockSpec((N, H2), lambda i: (0, 0))],
        out_specs=pl.BlockSpec((_OB, N), lambda i: (i, 0)),
        out_shape=jax.ShapeDtypeStruct((N, N), jnp.float32),
    )(z_p, z_p)

    return (pred_adj, mu_p[:N], lv_p[:N])
